# Initial kernel scaffold; baseline (speedup 1.0000x reference)
import jax
import jax.numpy as jnp
from jax.experimental import pallas as pl


def kernel(x, bn_gamma, bn_beta, W, in_idx, out_idx, kmap_sizes):
    raise NotImplementedError("write your pallas kernel here")


# speedup vs baseline: 10.0065x; 10.0065x over previous
import functools

import jax
import jax.numpy as jnp
from jax import lax
from jax.experimental import pallas as pl
from jax.experimental.pallas import tpu as pltpu
from jax.experimental.pallas import tpu_sc as plsc

_NC = 2
_NS = 16
_LANES = 16


def _stats_body(x_ref, s_ref):
    @pl.when(pl.program_id(0) == 0)
    def _():
        s_ref[...] = jnp.zeros_like(s_ref)

    xb = x_ref[...]
    s0 = jnp.sum(xb, axis=0)
    s1 = jnp.sum(xb * xb, axis=0)
    s_ref[...] += jnp.stack([s0, s1])


def _bn_stats(x, nblk):
    n, f = x.shape
    rows = n // nblk
    return pl.pallas_call(
        _stats_body,
        grid=(nblk,),
        in_specs=[pl.BlockSpec((rows, f), lambda i: (i, 0))],
        out_specs=pl.BlockSpec((2, f), lambda i: (0, 0)),
        out_shape=jax.ShapeDtypeStruct((2, f), jnp.float32),
    )(x)


def _norm_silu_body(n_rows, x_ref, s_ref, g_ref, b_ref, h_ref):
    s = s_ref[...]
    mean = s[0] / n_rows
    var = s[1] / n_rows - mean * mean
    scale = g_ref[0] * lax.rsqrt(var + 1e-5)
    shift = b_ref[0] - mean * scale
    t = x_ref[...] * scale + shift
    h_ref[...] = t * jax.nn.sigmoid(t)


def _norm_silu(x, sums, gamma, beta, nblk):
    n, f = x.shape
    rows = n // nblk
    return pl.pallas_call(
        functools.partial(_norm_silu_body, float(n)),
        grid=(nblk,),
        in_specs=[
            pl.BlockSpec((rows, f), lambda i: (i, 0)),
            pl.BlockSpec((2, f), lambda i: (0, 0)),
            pl.BlockSpec((1, f), lambda i: (0, 0)),
            pl.BlockSpec((1, f), lambda i: (0, 0)),
        ],
        out_specs=pl.BlockSpec((rows, f), lambda i: (i, 0)),
        out_shape=jax.ShapeDtypeStruct((n, f), jnp.float32),
    )(x, sums, gamma.reshape(1, f), beta.reshape(1, f))


def _sc_gather(h, in_pad, ep):
    n, f = h.shape
    chunk = 128
    per_w = ep // (_NC * _NS)
    iters = per_w // chunk
    mesh = plsc.VectorSubcoreMesh(core_axis_name="c", subcore_axis_name="s")

    @functools.partial(
        pl.kernel,
        mesh=mesh,
        out_type=jax.ShapeDtypeStruct((ep, f), jnp.float32),
        scratch_types=[
            pltpu.VMEM((chunk,), jnp.int32),
            pltpu.VMEM((chunk, f), jnp.float32),
            pltpu.SemaphoreType.DMA,
        ],
    )
    def gather_k(h_hbm, idx_hbm, out_hbm, idx_v, rows_v, sem):
        wid = lax.axis_index("s") * _NC + lax.axis_index("c")
        base = wid * per_w

        def body(i, _):
            off = base + i * chunk
            pltpu.sync_copy(idx_hbm.at[pl.ds(off, chunk)], idx_v)
            pltpu.async_copy(h_hbm.at[idx_v], rows_v, sem).wait()
            pltpu.sync_copy(rows_v, out_hbm.at[pl.ds(off, chunk)])
            return 0

        lax.fori_loop(0, iters, body, 0)

    return gather_k(h, in_pad)


def _mm_body(bk_ref, x_ref, w_ref, y_ref):
    del bk_ref
    y_ref[...] = jnp.dot(x_ref[...], w_ref[0],
                         preferred_element_type=jnp.float32)


def _grouped_matmul(h_src, w, block_k, blk):
    ep, f = h_src.shape
    fout = w.shape[-1]
    nb = ep // blk
    grid_spec = pltpu.PrefetchScalarGridSpec(
        num_scalar_prefetch=1,
        grid=(nb,),
        in_specs=[
            pl.BlockSpec((blk, f), lambda b, bk: (b, 0)),
            pl.BlockSpec((1, f, fout), lambda b, bk: (bk[b], 0, 0)),
        ],
        out_specs=pl.BlockSpec((blk, fout), lambda b, bk: (b, 0)),
    )
    return pl.pallas_call(
        _mm_body,
        grid_spec=grid_spec,
        out_shape=jax.ShapeDtypeStruct((ep, fout), jnp.float32),
    )(block_k, h_src, w)


def _sc_scatter_add(y, out_pad, n_pad):
    ep, f = y.shape
    chunk = 128
    nranges = 4
    nr = n_pad // nranges
    region = 13312
    dump = region - 1
    zrows = 208
    och = 200
    per_s = ep // _NS
    iters = per_s // chunk
    zeros = jnp.zeros((zrows, f), jnp.float32)
    mesh = plsc.VectorSubcoreMesh(core_axis_name="c", subcore_axis_name="s")

    @functools.partial(
        pl.kernel,
        mesh=mesh,
        out_type=jax.ShapeDtypeStruct((n_pad, f), jnp.float32),
        scratch_types=[
            pltpu.VMEM((chunk,), jnp.int32),
            pltpu.VMEM((chunk,), jnp.int32),
            pltpu.VMEM((chunk, f), jnp.float32),
            pltpu.VMEM_SHARED((region, f), jnp.float32),
        ],
    )
    def scatter_k(y_hbm, opad_hbm, z_hbm, out_hbm, idx_raw, idx_loc,
                  rows_v, shared):
        c = lax.axis_index("c")
        s = lax.axis_index("s")

        for j in range(nranges // _NC):
            r = c * (nranges // _NC) + j
            r_base = r * nr

            for z in range(region // _NS // zrows):
                pltpu.sync_copy(
                    z_hbm, shared.at[pl.ds(s * (region // _NS) + z * zrows,
                                           zrows)])
            plsc.subcore_barrier()

            def chunk_body(i, _):
                off = s * per_s + i * chunk
                pltpu.sync_copy(opad_hbm.at[pl.ds(off, chunk)], idx_raw)

                def vec_body(v, _):
                    d = idx_raw[pl.ds(v * _LANES, _LANES)]
                    lo = d - r_base
                    ok = (lo >= 0) & (lo < nr)
                    idx_loc[pl.ds(v * _LANES, _LANES)] = jnp.where(
                        ok, lo, dump)
                    return 0

                lax.fori_loop(0, chunk // _LANES, vec_body, 0)
                pltpu.sync_copy(y_hbm.at[pl.ds(off, chunk)], rows_v)
                pltpu.sync_copy(rows_v, shared.at[idx_loc], add=True)
                return 0

            lax.fori_loop(0, iters, chunk_body, 0)
            plsc.subcore_barrier()

            for oc in range(nr // _NS // och):
                row = s * (nr // _NS) + oc * och
                pltpu.sync_copy(shared.at[pl.ds(row, och)],
                                out_hbm.at[pl.ds(r_base + row, och)])
            plsc.subcore_barrier()

    return scatter_k(y, out_pad, zeros)


def kernel(x, bn_gamma, bn_beta, W, in_idx, out_idx, kmap_sizes):
    n, f = x.shape
    kvol, _, fout = W.shape
    e = in_idx.shape[0]
    blk = 256
    ep = ((e + kvol * (blk - 1) + 4095) // 4096) * 4096

    nblk = 25
    sums = _bn_stats(x, nblk)
    h = _norm_silu(x, sums, bn_gamma, bn_beta, nblk)

    sizes = kmap_sizes.astype(jnp.int32)
    csum = jnp.cumsum(sizes)
    cexcl = csum - sizes
    padded = ((sizes + blk - 1) // blk) * blk
    ostart = jnp.cumsum(padded) - padded
    p = jnp.arange(ep, dtype=jnp.int32)
    slot_k = jnp.clip(
        jnp.searchsorted(ostart, p, side="right").astype(jnp.int32) - 1,
        0, kvol - 1)
    rel = p - ostart[slot_k]
    edge = rel + cexcl[slot_k]
    valid = rel < sizes[slot_k]
    ec = jnp.clip(edge, 0, e - 1)
    in_pad = jnp.where(valid, in_idx[ec], 0).astype(jnp.int32)
    out_pad = jnp.where(valid, out_idx[ec], -1).astype(jnp.int32)
    block_k = jnp.clip(
        jnp.searchsorted(ostart,
                         jnp.arange(ep // blk, dtype=jnp.int32) * blk,
                         side="right").astype(jnp.int32) - 1,
        0, kvol - 1)

    h_src = _sc_gather(h, in_pad, ep)
    y = _grouped_matmul(h_src, W, block_k, blk)
    n_pad = ((n + 4 * _NS * 200 - 1) // (4 * _NS * 200)) * (4 * _NS * 200)
    out = _sc_scatter_add(y, out_pad, n_pad)
    return out[:n]


# speedup vs baseline: 10.3755x; 1.0369x over previous
import functools

import jax
import jax.numpy as jnp
from jax import lax
from jax.experimental import pallas as pl
from jax.experimental.pallas import tpu as pltpu
from jax.experimental.pallas import tpu_sc as plsc

_NC = 2
_NS = 16
_LANES = 16


def _stats_body(x_ref, s_ref):
    @pl.when(pl.program_id(0) == 0)
    def _():
        s_ref[...] = jnp.zeros_like(s_ref)

    xb = x_ref[...]
    s0 = jnp.sum(xb, axis=0)
    s1 = jnp.sum(xb * xb, axis=0)
    s_ref[...] += jnp.stack([s0, s1])


def _bn_stats(x, nblk):
    n, f = x.shape
    rows = n // nblk
    return pl.pallas_call(
        _stats_body,
        grid=(nblk,),
        in_specs=[pl.BlockSpec((rows, f), lambda i: (i, 0))],
        out_specs=pl.BlockSpec((2, f), lambda i: (0, 0)),
        out_shape=jax.ShapeDtypeStruct((2, f), jnp.float32),
    )(x)


def _norm_silu_body(n_rows, x_ref, s_ref, g_ref, b_ref, h_ref):
    s = s_ref[...]
    mean = s[0] / n_rows
    var = s[1] / n_rows - mean * mean
    scale = g_ref[0] * lax.rsqrt(var + 1e-5)
    shift = b_ref[0] - mean * scale
    t = x_ref[...] * scale + shift
    h_ref[...] = t * jax.nn.sigmoid(t)


def _norm_silu(x, sums, gamma, beta, nblk):
    n, f = x.shape
    rows = n // nblk
    return pl.pallas_call(
        functools.partial(_norm_silu_body, float(n)),
        grid=(nblk,),
        in_specs=[
            pl.BlockSpec((rows, f), lambda i: (i, 0)),
            pl.BlockSpec((2, f), lambda i: (0, 0)),
            pl.BlockSpec((1, f), lambda i: (0, 0)),
            pl.BlockSpec((1, f), lambda i: (0, 0)),
        ],
        out_specs=pl.BlockSpec((rows, f), lambda i: (i, 0)),
        out_shape=jax.ShapeDtypeStruct((n, f), jnp.float32),
    )(x, sums, gamma.reshape(1, f), beta.reshape(1, f))


def _sc_gather(h, in_pad, ep):
    n, f = h.shape
    chunk = 128
    nbuf = 4
    per_w = ep // (_NC * _NS)
    iters = per_w // chunk
    mesh = plsc.VectorSubcoreMesh(core_axis_name="c", subcore_axis_name="s")

    @functools.partial(
        pl.kernel,
        mesh=mesh,
        out_type=jax.ShapeDtypeStruct((ep, f), jnp.float32),
        scratch_types=[
            pltpu.VMEM((per_w,), jnp.int32),
            *[pltpu.VMEM((chunk, f), jnp.float32) for _ in range(nbuf)],
            *[pltpu.SemaphoreType.DMA for _ in range(2 * nbuf)],
        ],
    )
    def gather_k(h_hbm, idx_hbm, out_hbm, idx_all, *bufs_sems):
        rows = bufs_sems[:nbuf]
        gsem = bufs_sems[nbuf:2 * nbuf]
        ssem = bufs_sems[2 * nbuf:]
        wid = lax.axis_index("s") * _NC + lax.axis_index("c")
        base = wid * per_w

        pltpu.sync_copy(idx_hbm.at[pl.ds(base, per_w)], idx_all)

        depth = nbuf - 1
        ghandles = [None] * nbuf
        shandles = [None] * nbuf

        def issue_gather(j):
            p = j % nbuf
            ghandles[p] = pltpu.async_copy(
                h_hbm.at[idx_all.at[pl.ds(j * chunk, chunk)]],
                rows[p], gsem[p])

        for j in range(min(depth, iters)):
            issue_gather(j)
        for i in range(iters):
            p = i % nbuf
            ghandles[p].wait()
            shandles[p] = pltpu.async_copy(
                rows[p], out_hbm.at[pl.ds(base + i * chunk, chunk)], ssem[p])
            j = i + depth
            if j < iters:
                pj = j % nbuf
                if shandles[pj] is not None:
                    shandles[pj].wait()
                issue_gather(j)
        for p in range(nbuf):
            if shandles[p] is not None:
                shandles[p].wait()

    return gather_k(h, in_pad)


def _mm_body(bk_ref, x_ref, w_ref, y_ref):
    del bk_ref
    y_ref[...] = jnp.dot(x_ref[...], w_ref[0],
                         preferred_element_type=jnp.float32)


def _grouped_matmul(h_src, w, block_k, blk):
    ep, f = h_src.shape
    fout = w.shape[-1]
    nb = ep // blk
    grid_spec = pltpu.PrefetchScalarGridSpec(
        num_scalar_prefetch=1,
        grid=(nb,),
        in_specs=[
            pl.BlockSpec((blk, f), lambda b, bk: (b, 0)),
            pl.BlockSpec((1, f, fout), lambda b, bk: (bk[b], 0, 0)),
        ],
        out_specs=pl.BlockSpec((blk, fout), lambda b, bk: (b, 0)),
    )
    return pl.pallas_call(
        _mm_body,
        grid_spec=grid_spec,
        out_shape=jax.ShapeDtypeStruct((ep, fout), jnp.float32),
    )(block_k, h_src, w)


def _sc_scatter_add(y, out_pad, n_pad):
    ep, f = y.shape
    chunk = 32
    nranges = 4
    nr = n_pad // nranges
    region = nr + 8
    dump = nr
    nch = nr // chunk
    per_s = ep // _NS
    iters = per_s // chunk
    nbuf = 4
    outer = iters // nbuf
    mesh = plsc.VectorSubcoreMesh(core_axis_name="c", subcore_axis_name="s")

    @functools.partial(
        pl.kernel,
        mesh=mesh,
        out_type=jax.ShapeDtypeStruct((n_pad, f), jnp.float32),
        scratch_types=[
            pltpu.VMEM((nbuf, chunk), jnp.int32),
            pltpu.VMEM((nbuf, chunk), jnp.int32),
            pltpu.VMEM_SHARED((region, f), jnp.float32),
            *[pltpu.VMEM((chunk, f), jnp.float32) for _ in range(nbuf)],
            *[pltpu.SemaphoreType.DMA for _ in range(2 * nbuf)],
        ],
    )
    def scatter_k(y_hbm, opad_hbm, out_hbm, idx_raw, idx_loc, shared,
                  *bufs_sems):
        rows = bufs_sems[:nbuf]
        lsem = bufs_sems[nbuf:2 * nbuf]
        isem = bufs_sems[2 * nbuf:]
        c = lax.axis_index("c")
        s = lax.axis_index("s")
        base = s * per_s

        for j in range(nranges // _NC):
            r = c * (nranges // _NC) + j
            r_base = r * nr

            def zero_body(i, _):
                for v in range(f // _LANES):
                    rows[0][i, pl.ds(v * _LANES, _LANES)] = jnp.zeros(
                        (_LANES,), jnp.float32)
                return 0

            lax.fori_loop(0, chunk, zero_body, 0)
            for t in range((nch + _NS - 1) // _NS):
                cid = t * _NS + s

                @pl.when(cid < nch)
                def _():
                    pltpu.sync_copy(rows[0],
                                    shared.at[pl.ds(cid * chunk, chunk)])
            plsc.subcore_barrier()

            def issue_load(k, p):
                pltpu.async_copy(
                    opad_hbm.at[pl.ds(base + k * chunk, chunk)],
                    idx_raw.at[p], isem[p])
                pltpu.async_copy(
                    y_hbm.at[pl.ds(base + k * chunk, chunk)],
                    rows[p], lsem[p])

            for b in range(nbuf):
                issue_load(b, b)

            def ring_body(g, _):
                for b in range(nbuf):
                    pltpu.make_async_copy(
                        opad_hbm.at[pl.ds(0, chunk)], idx_raw.at[b],
                        isem[b]).wait()
                    for v in range(chunk // _LANES):
                        d = idx_raw[b, pl.ds(v * _LANES, _LANES)]
                        lo = d - r_base
                        ok = (lo >= 0) & (lo < nr)
                        idx_loc[b, pl.ds(v * _LANES, _LANES)] = jnp.where(
                            ok, lo, dump)
                    pltpu.make_async_copy(
                        y_hbm.at[pl.ds(0, chunk)], rows[b],
                        lsem[b]).wait()
                    pltpu.sync_copy(rows[b], shared.at[idx_loc.at[b]],
                                    add=True)

                    @pl.when(g < outer - 1)
                    def _():
                        issue_load((g + 1) * nbuf + b, b)
                return 0

            lax.fori_loop(0, outer, ring_body, 0)
            plsc.subcore_barrier()

            for t in range((nch + _NS - 1) // _NS):
                cid = t * _NS + s

                @pl.when(cid < nch)
                def _():
                    pltpu.sync_copy(
                        shared.at[pl.ds(cid * chunk, chunk)],
                        out_hbm.at[pl.ds(r_base + cid * chunk, chunk)])
            plsc.subcore_barrier()

    return scatter_k(y, out_pad)


def kernel(x, bn_gamma, bn_beta, W, in_idx, out_idx, kmap_sizes):
    n, f = x.shape
    kvol, _, fout = W.shape
    e = in_idx.shape[0]
    blk = 256
    ep = ((e + kvol * (blk - 1) + 4095) // 4096) * 4096

    nblk = 25
    sums = _bn_stats(x, nblk)
    h = _norm_silu(x, sums, bn_gamma, bn_beta, nblk)

    sizes = kmap_sizes.astype(jnp.int32)
    csum = jnp.cumsum(sizes)
    cexcl = csum - sizes
    padded = ((sizes + blk - 1) // blk) * blk
    ostart = jnp.cumsum(padded) - padded
    p = jnp.arange(ep, dtype=jnp.int32)
    slot_k = jnp.clip(
        jnp.searchsorted(ostart, p, side="right").astype(jnp.int32) - 1,
        0, kvol - 1)
    rel = p - ostart[slot_k]
    edge = rel + cexcl[slot_k]
    valid = rel < sizes[slot_k]
    ec = jnp.clip(edge, 0, e - 1)
    in_pad = jnp.where(valid, in_idx[ec], 0).astype(jnp.int32)
    out_pad = jnp.where(valid, out_idx[ec], -1).astype(jnp.int32)
    block_k = jnp.clip(
        jnp.searchsorted(ostart,
                         jnp.arange(ep // blk, dtype=jnp.int32) * blk,
                         side="right").astype(jnp.int32) - 1,
        0, kvol - 1)

    h_src = _sc_gather(h, in_pad, ep)
    y = _grouped_matmul(h_src, W, block_k, blk)
    nranges = 4
    nr = (((n + nranges - 1) // nranges + 127) // 128) * 128
    out = _sc_scatter_add(y, out_pad, nranges * nr)
    return out[:n]
